# baseline (speedup 1.0000x reference)
import functools

import jax
import jax.numpy as jnp
from jax import lax
from jax.experimental import pallas as pl
from jax.experimental.pallas import tpu as pltpu
from jax.experimental.pallas import tpu_sc as plsc

_B, _N, _D = 1024, 200, 64
_NC, _NS = 2, 16
_NW = _NC * _NS
_BPW = _B // _NW
_C0, _C1 = 128, 72


@functools.partial(
    pl.kernel,
    out_type=jax.ShapeDtypeStruct((_B, _N, 2 * _D), jnp.float32),
    mesh=plsc.VectorSubcoreMesh(core_axis_name="c", subcore_axis_name="s"),
    compiler_params=pltpu.CompilerParams(use_tc_tiling_on_sc=False),
    scratch_types=[
        pltpu.VMEM((_N,), jnp.int32),
        pltpu.VMEM((_N, _D), jnp.float32),
        pltpu.SemaphoreType.DMA,
    ],
)
def _sc_concat_gather(dis_hbm, men_hbm, idx_hbm, out_hbm, idx_v, rows_v, sem):
  wid = lax.axis_index("s") * _NC + lax.axis_index("c")

  def body(i, carry):
    b = wid * _BPW + i
    pltpu.sync_copy(idx_hbm.at[b], idx_v)
    cp0 = pltpu.async_copy(
        men_hbm.at[b].at[idx_v.at[pl.ds(0, _C0)]],
        rows_v.at[pl.ds(0, _C0)], sem)
    cp1 = pltpu.async_copy(
        men_hbm.at[b].at[idx_v.at[pl.ds(_C0, _C1)]],
        rows_v.at[pl.ds(_C0, _C1)], sem)
    cp0.wait()
    cp1.wait()
    pltpu.sync_copy(rows_v, out_hbm.at[b, :, pl.ds(0, _D)])
    pltpu.sync_copy(dis_hbm.at[b], out_hbm.at[b, :, pl.ds(_D, _D)])
    return carry

  lax.fori_loop(0, _BPW, body, 0)


def kernel(dis_entity, men_state, dis_entity_mark):
  return _sc_concat_gather(dis_entity, men_state,
                           dis_entity_mark.astype(jnp.int32))


# speedup vs baseline: 1.0002x; 1.0002x over previous
import functools

import jax
import jax.numpy as jnp
from jax import lax
from jax.experimental import pallas as pl
from jax.experimental.pallas import tpu as pltpu
from jax.experimental.pallas import tpu_sc as plsc

_B, _N, _D = 1024, 200, 64
_NC, _NS = 2, 16
_NW = _NC * _NS
_BPW = _B // _NW
_C0, _C1 = 128, 72
_LOOK = 4
_NB = 8


@functools.partial(
    pl.kernel,
    out_type=jax.ShapeDtypeStruct((_B, _N, 2 * _D), jnp.float32),
    mesh=plsc.VectorSubcoreMesh(core_axis_name="c", subcore_axis_name="s"),
    compiler_params=pltpu.CompilerParams(use_tc_tiling_on_sc=False),
    scratch_types=(
        [pltpu.VMEM((_BPW, _N), jnp.int32)]
        + [pltpu.VMEM((_N, _D), jnp.float32) for _ in range(_NB)]
        + [pltpu.SemaphoreType.DMA for _ in range(2 * _NB + 1)]
    ),
)
def _sc_concat_gather(dis_hbm, men_hbm, idx_hbm, out_hbm, idx_v, *scratch):
  rows = scratch[:_NB]
  gsems = scratch[_NB:2 * _NB]
  wsems = scratch[2 * _NB:3 * _NB]
  dsem = scratch[3 * _NB]

  wid = lax.axis_index("s") * _NC + lax.axis_index("c")
  base = wid * _BPW
  pltpu.sync_copy(idx_hbm.at[pl.ds(base, _BPW)], idx_v)

  dis_cps = [
      pltpu.async_copy(dis_hbm.at[base + i],
                       out_hbm.at[base + i, :, pl.ds(_D, _D)], dsem)
      for i in range(_BPW)
  ]

  def fire(i):
    b = base + i
    buf, gsem = rows[i % _NB], gsems[i % _NB]
    c0 = pltpu.async_copy(men_hbm.at[b].at[idx_v.at[i, pl.ds(0, _C0)]],
                          buf.at[pl.ds(0, _C0)], gsem)
    c1 = pltpu.async_copy(men_hbm.at[b].at[idx_v.at[i, pl.ds(_C0, _C1)]],
                          buf.at[pl.ds(_C0, _C1)], gsem)
    return (c0, c1)

  gath = [None] * _NB
  writes = [None] * _NB
  for i in range(_LOOK):
    gath[i % _NB] = fire(i)
  for i in range(_BPW):
    j = i + _LOOK
    if j < _BPW:
      nb = j % _NB
      if writes[nb] is not None:
        writes[nb].wait()
      gath[nb] = fire(j)
    for c in gath[i % _NB]:
      c.wait()
    writes[i % _NB] = pltpu.async_copy(
        rows[i % _NB], out_hbm.at[base + i, :, pl.ds(0, _D)], wsems[i % _NB])
  for w in writes:
    if w is not None:
      w.wait()
  for c in dis_cps:
    c.wait()


def kernel(dis_entity, men_state, dis_entity_mark):
  return _sc_concat_gather(dis_entity, men_state,
                           dis_entity_mark.astype(jnp.int32))


# speedup vs baseline: 5.7165x; 5.7154x over previous
import functools

import jax
import jax.numpy as jnp
from jax import lax
from jax.experimental import pallas as pl
from jax.experimental.pallas import tpu as pltpu
from jax.experimental.pallas import tpu_sc as plsc

_B, _N, _D = 1024, 200, 64
_NC, _NS = 2, 16
_NW = _NC * _NS
_BPW = _B // _NW
_C0, _C1 = 128, 72
_LOOK = 2
_NB = 4


@functools.partial(
    pl.kernel,
    out_type=jax.ShapeDtypeStruct((_B, _N, 2 * _D), jnp.float32),
    mesh=plsc.VectorSubcoreMesh(core_axis_name="c", subcore_axis_name="s"),
    compiler_params=pltpu.CompilerParams(use_tc_tiling_on_sc=False),
    scratch_types=(
        [pltpu.VMEM((_BPW, _N), jnp.int32)]
        + [pltpu.VMEM((_N, _D), jnp.float32) for _ in range(2 * _NB)]
        + [pltpu.SemaphoreType.DMA for _ in range(2 * _NB)]
    ),
)
def _sc_concat_gather(dis_hbm, men_hbm, idx_hbm, out_hbm, idx_v, *scratch):
  rows = scratch[:_NB]
  disb = scratch[_NB:2 * _NB]
  gsems = scratch[2 * _NB:3 * _NB]
  wsems = scratch[3 * _NB:4 * _NB]

  wid = lax.axis_index("s") * _NC + lax.axis_index("c")
  base = wid * _BPW
  pltpu.sync_copy(idx_hbm.at[pl.ds(base, _BPW)], idx_v)

  def fire(j):
    slot = j % _NB
    b = base + j
    c0 = pltpu.async_copy(men_hbm.at[b].at[idx_v.at[j, pl.ds(0, _C0)]],
                          rows[slot].at[pl.ds(0, _C0)], gsems[slot])
    c1 = pltpu.async_copy(men_hbm.at[b].at[idx_v.at[j, pl.ds(_C0, _C1)]],
                          rows[slot].at[pl.ds(_C0, _C1)], gsems[slot])
    c2 = pltpu.async_copy(dis_hbm.at[b], disb[slot], gsems[slot])
    return (c0, c1, c2)

  gath = [None] * _NB
  wr = [None] * _NB
  for j in range(_LOOK):
    gath[j % _NB] = fire(j)
  for i in range(_BPW):
    j = i + _LOOK
    if j < _BPW:
      slot = j % _NB
      if wr[slot] is not None:
        for w in wr[slot]:
          w.wait()
      gath[slot] = fire(j)
    slot = i % _NB
    for c in gath[slot]:
      c.wait()
    b = base + i
    w0 = pltpu.async_copy(rows[slot], out_hbm.at[b, :, pl.ds(0, _D)],
                          wsems[slot])
    w1 = pltpu.async_copy(disb[slot], out_hbm.at[b, :, pl.ds(_D, _D)],
                          wsems[slot])
    wr[slot] = (w0, w1)
  for ws in wr:
    if ws is not None:
      for w in ws:
        w.wait()


def kernel(dis_entity, men_state, dis_entity_mark):
  return _sc_concat_gather(dis_entity, men_state,
                           dis_entity_mark.astype(jnp.int32))


# speedup vs baseline: 5.7292x; 1.0022x over previous
import functools

import jax
import jax.numpy as jnp
from jax import lax
from jax.experimental import pallas as pl
from jax.experimental.pallas import tpu as pltpu
from jax.experimental.pallas import tpu_sc as plsc

_B, _N, _D = 1024, 200, 64
_NC, _NS = 2, 16
_NW = _NC * _NS
_BPW = _B // _NW
_C0, _C1 = 128, 72
_LOOK = 2
_NB = 4
_NCH = 13


@functools.partial(
    pl.kernel,
    out_type=jax.ShapeDtypeStruct((_B, _N, 2 * _D), jnp.float32),
    mesh=plsc.VectorSubcoreMesh(core_axis_name="c", subcore_axis_name="s"),
    compiler_params=pltpu.CompilerParams(use_tc_tiling_on_sc=False,
                                         needs_layout_passes=False),
    scratch_types=(
        [pltpu.VMEM((_N, _BPW), jnp.int32)]
        + [pltpu.VMEM((_N,), jnp.int32) for _ in range(_NB)]
        + [pltpu.VMEM((_N, _D), jnp.float32) for _ in range(2 * _NB)]
        + [pltpu.SemaphoreType.DMA for _ in range(2 * _NB)]
    ),
)
def _sc_concat_gather(dis_hbm, men_hbm, idxt_hbm, out_hbm, idx_v, *scratch):
  lists = scratch[:_NB]
  rows = scratch[_NB:2 * _NB]
  disb = scratch[2 * _NB:3 * _NB]
  gsems = scratch[3 * _NB:4 * _NB]
  wsems = scratch[4 * _NB:5 * _NB]

  wid = lax.axis_index("s") * _NC + lax.axis_index("c")
  base = wid * _BPW
  pltpu.sync_copy(idxt_hbm.at[:, pl.ds(base, _BPW)], idx_v)

  def build_list(j):
    slot = j % _NB
    col = jnp.full((16,), j, jnp.int32)
    for k in range(_NCH):
      r0 = min(16 * k, _N - 16)
      rvec = lax.iota(jnp.int32, 16) + r0
      vals = plsc.load_gather(idx_v, [rvec, col])
      lists[slot][pl.ds(r0, 16)] = vals

  def fire(j):
    slot = j % _NB
    b = base + j
    c0 = pltpu.async_copy(men_hbm.at[b].at[lists[slot].at[pl.ds(0, _C0)]],
                          rows[slot].at[pl.ds(0, _C0)], gsems[slot])
    c1 = pltpu.async_copy(men_hbm.at[b].at[lists[slot].at[pl.ds(_C0, _C1)]],
                          rows[slot].at[pl.ds(_C0, _C1)], gsems[slot])
    c2 = pltpu.async_copy(dis_hbm.at[b], disb[slot], gsems[slot])
    return (c0, c1, c2)

  gath = [None] * _NB
  wr = [None] * _NB
  for j in range(_LOOK):
    build_list(j)
    gath[j % _NB] = fire(j)
  for i in range(_BPW):
    j = i + _LOOK
    if j < _BPW:
      slot = j % _NB
      if wr[slot] is not None:
        for w in wr[slot]:
          w.wait()
      build_list(j)
      gath[slot] = fire(j)
    slot = i % _NB
    for c in gath[slot]:
      c.wait()
    b = base + i
    w0 = pltpu.async_copy(rows[slot], out_hbm.at[b, :, pl.ds(0, _D)],
                          wsems[slot])
    w1 = pltpu.async_copy(disb[slot], out_hbm.at[b, :, pl.ds(_D, _D)],
                          wsems[slot])
    wr[slot] = (w0, w1)
  for ws in wr:
    if ws is not None:
      for w in ws:
        w.wait()


def kernel(dis_entity, men_state, dis_entity_mark):
  idx_t = dis_entity_mark.astype(jnp.int32).T
  return _sc_concat_gather(dis_entity, men_state, idx_t)


# speedup vs baseline: 5.7428x; 1.0024x over previous
import functools

import jax
import jax.numpy as jnp
from jax import lax
from jax.experimental import pallas as pl
from jax.experimental.pallas import tpu as pltpu
from jax.experimental.pallas import tpu_sc as plsc

_B, _N, _D = 1024, 200, 64
_NC, _NS = 2, 16
_NW = _NC * _NS
_BPW = _B // _NW
_C0, _C1 = 128, 72
_LOOK = 2
_NB = 4
_NCH = 13


@functools.partial(
    pl.kernel,
    out_type=jax.ShapeDtypeStruct((_B, _N, 2 * _D), jnp.float32),
    mesh=plsc.VectorSubcoreMesh(core_axis_name="c", subcore_axis_name="s"),
    compiler_params=pltpu.CompilerParams(use_tc_tiling_on_sc=False,
                                         needs_layout_passes=False),
    scratch_types=(
        [pltpu.VMEM((_BPW, _N), jnp.float32)]
        + [pltpu.VMEM((_N,), jnp.int32) for _ in range(_NB)]
        + [pltpu.VMEM((_N, _D), jnp.float32) for _ in range(2 * _NB)]
        + [pltpu.SemaphoreType.DMA for _ in range(2 * _NB)]
    ),
)
def _sc_concat_gather(dis_hbm, men_hbm, idx_hbm, out_hbm, idx_v, *scratch):
  lists = scratch[:_NB]
  rows = scratch[_NB:2 * _NB]
  disb = scratch[2 * _NB:3 * _NB]
  gsems = scratch[3 * _NB:4 * _NB]
  wsems = scratch[4 * _NB:5 * _NB]

  wid = lax.axis_index("s") * _NC + lax.axis_index("c")
  base = wid * _BPW
  pltpu.sync_copy(idx_hbm.at[pl.ds(base, _BPW)], idx_v)

  def build_list(j):
    slot = j % _NB
    for k in range(_NCH):
      r0 = min(16 * k, _N - 16)
      vals = plsc.bitcast(idx_v[j, pl.ds(r0, 16)], jnp.int32)
      lists[slot][pl.ds(r0, 16)] = vals

  def fire(j):
    slot = j % _NB
    b = base + j
    c0 = pltpu.async_copy(men_hbm.at[b].at[lists[slot].at[pl.ds(0, _C0)]],
                          rows[slot].at[pl.ds(0, _C0)], gsems[slot])
    c1 = pltpu.async_copy(men_hbm.at[b].at[lists[slot].at[pl.ds(_C0, _C1)]],
                          rows[slot].at[pl.ds(_C0, _C1)], gsems[slot])
    c2 = pltpu.async_copy(dis_hbm.at[b], disb[slot], gsems[slot])
    return (c0, c1, c2)

  gath = [None] * _NB
  wr = [None] * _NB
  for j in range(_LOOK):
    build_list(j)
    gath[j % _NB] = fire(j)
  for i in range(_BPW):
    j = i + _LOOK
    if j < _BPW:
      slot = j % _NB
      if wr[slot] is not None:
        for w in wr[slot]:
          w.wait()
      build_list(j)
      gath[slot] = fire(j)
    slot = i % _NB
    for c in gath[slot]:
      c.wait()
    b = base + i
    w0 = pltpu.async_copy(rows[slot], out_hbm.at[b, :, pl.ds(0, _D)],
                          wsems[slot])
    w1 = pltpu.async_copy(disb[slot], out_hbm.at[b, :, pl.ds(_D, _D)],
                          wsems[slot])
    wr[slot] = (w0, w1)
  for ws in wr:
    if ws is not None:
      for w in ws:
        w.wait()


def kernel(dis_entity, men_state, dis_entity_mark):
  idx_f = lax.bitcast_convert_type(dis_entity_mark.astype(jnp.int32),
                                   jnp.float32)
  return _sc_concat_gather(dis_entity, men_state, idx_f)


# speedup vs baseline: 5.8909x; 1.0258x over previous
import functools

import jax
import jax.numpy as jnp
from jax import lax
from jax.experimental import pallas as pl
from jax.experimental.pallas import tpu as pltpu
from jax.experimental.pallas import tpu_sc as plsc

_B, _N, _D = 1024, 200, 64
_NC, _NS = 2, 16
_NW = _NC * _NS
_BPW = _B // _NW
_C0, _C1 = 128, 72
_LOOK = 2
_NB = 4
_NCH = 13

_MESH = plsc.VectorSubcoreMesh(core_axis_name="c", subcore_axis_name="s")
_PARAMS = pltpu.CompilerParams(use_tc_tiling_on_sc=False,
                               needs_layout_passes=False)


def _worker_base():
  wid = lax.axis_index("s") * _NC + lax.axis_index("c")
  return wid * _BPW


@functools.partial(
    pl.kernel, mesh=_MESH, compiler_params=_PARAMS,
    scratch_types=(
        [pltpu.VMEM((_N, _D), jnp.float32) for _ in range(_NB)]
        + [pltpu.SemaphoreType.DMA for _ in range(2 * _NB)]
    ),
)
def _sc_dis(dis_hbm, out_hbm, *scratch):
  disb = scratch[:_NB]
  gsems = scratch[_NB:2 * _NB]
  wsems = scratch[2 * _NB:3 * _NB]
  base = _worker_base()

  gath = [None] * _NB
  wr = [None] * _NB
  for j in range(_LOOK):
    gath[j % _NB] = pltpu.async_copy(dis_hbm.at[base + j], disb[j % _NB],
                                     gsems[j % _NB])
  for i in range(_BPW):
    j = i + _LOOK
    if j < _BPW:
      slot = j % _NB
      if wr[slot] is not None:
        wr[slot].wait()
      gath[slot] = pltpu.async_copy(dis_hbm.at[base + j], disb[slot],
                                    gsems[slot])
    slot = i % _NB
    gath[slot].wait()
    wr[slot] = pltpu.async_copy(disb[slot],
                                out_hbm.at[base + i, :, pl.ds(_D, _D)],
                                wsems[slot])
  for w in wr:
    if w is not None:
      w.wait()


@functools.partial(
    pl.kernel, mesh=_MESH, compiler_params=_PARAMS,
    scratch_types=(
        [pltpu.VMEM((_BPW, _N), jnp.float32)]
        + [pltpu.VMEM((_N,), jnp.int32) for _ in range(_NB)]
        + [pltpu.VMEM((_N, _D), jnp.float32) for _ in range(_NB)]
        + [pltpu.SemaphoreType.DMA for _ in range(2 * _NB)]
    ),
)
def _sc_men(men_hbm, idx_hbm, out_hbm, idx_v, *scratch):
  lists = scratch[:_NB]
  rows = scratch[_NB:2 * _NB]
  gsems = scratch[2 * _NB:3 * _NB]
  wsems = scratch[3 * _NB:4 * _NB]
  base = _worker_base()
  pltpu.sync_copy(idx_hbm.at[pl.ds(base, _BPW)], idx_v)

  def build_list(j):
    slot = j % _NB
    for k in range(_NCH):
      r0 = min(16 * k, _N - 16)
      lists[slot][pl.ds(r0, 16)] = plsc.bitcast(idx_v[j, pl.ds(r0, 16)],
                                                jnp.int32)

  def fire(j):
    slot = j % _NB
    b = base + j
    c0 = pltpu.async_copy(men_hbm.at[b].at[lists[slot].at[pl.ds(0, _C0)]],
                          rows[slot].at[pl.ds(0, _C0)], gsems[slot])
    c1 = pltpu.async_copy(men_hbm.at[b].at[lists[slot].at[pl.ds(_C0, _C1)]],
                          rows[slot].at[pl.ds(_C0, _C1)], gsems[slot])
    return (c0, c1)

  gath = [None] * _NB
  wr = [None] * _NB
  for j in range(_LOOK):
    build_list(j)
    gath[j % _NB] = fire(j)
  for i in range(_BPW):
    j = i + _LOOK
    if j < _BPW:
      slot = j % _NB
      if wr[slot] is not None:
        wr[slot].wait()
      build_list(j)
      gath[slot] = fire(j)
    slot = i % _NB
    for c in gath[slot]:
      c.wait()
    wr[slot] = pltpu.async_copy(rows[slot],
                                out_hbm.at[base + i, :, pl.ds(0, _D)],
                                wsems[slot])
  for w in wr:
    if w is not None:
      w.wait()


def kernel(dis_entity, men_state, dis_entity_mark):
  idx_f = lax.bitcast_convert_type(dis_entity_mark.astype(jnp.int32),
                                   jnp.float32)
  out_ref = jax.empty_ref(
      jax.ShapeDtypeStruct((_B, _N, 2 * _D), jnp.float32))
  _sc_dis(dis_entity, out_ref)
  _sc_men(men_state, idx_f, out_ref)
  return out_ref[...]
